# baseline (speedup 1.0000x reference)
import functools

import jax
import jax.numpy as jnp
from jax import lax
from jax.experimental import pallas as pl
from jax.experimental.pallas import tpu as pltpu
from jax.experimental.pallas import tpu_sc as plsc

N = 10000
E = 320000
DF = 128
NPAD = 10240
EPAD = 327680
NW = 32
NSUB = 16
CH = 128
EPW = EPAD // NW
NCH = EPW // CH
TOTCH = EPAD // CH
G0 = 154
G1 = 6
SPT = NPAD // NSUB
DUMMY = 10200
BR = 1000
GRID = N // BR


def _sc_mesh():
    return plsc.VectorSubcoreMesh(core_axis_name="c", subcore_axis_name="s")


def _deg_body(dst_hbm, zeros_hbm, ones_hbm, out_hbm, didx2, ones_v, deg_sh):
    c = lax.axis_index("c")
    s = lax.axis_index("s")
    wid = c * NSUB + s
    pltpu.sync_copy(zeros_hbm.at[pl.ds(s * SPT, SPT)], deg_sh.at[pl.ds(s * SPT, SPT)])
    pltpu.sync_copy(dst_hbm.at[wid], didx2)
    pltpu.sync_copy(ones_hbm.at[pl.ds(0, CH)], ones_v)
    plsc.subcore_barrier()

    def body(i, carry):
        pltpu.sync_copy(ones_v, deg_sh.at[didx2.at[i]], add=True)
        return carry

    lax.fori_loop(0, NCH, body, 0)
    plsc.subcore_barrier()
    pltpu.sync_copy(deg_sh.at[pl.ds(s * SPT, SPT)], out_hbm.at[c, pl.ds(s * SPT, SPT)])


def _sc_degree(dst3, zeros128, ones128):
    return pl.kernel(
        _deg_body,
        out_type=jax.ShapeDtypeStruct((2, NPAD, DF), jnp.float32),
        mesh=_sc_mesh(),
        scratch_types=[
            pltpu.VMEM((NCH, CH), jnp.int32),
            pltpu.VMEM((CH, DF), jnp.float32),
            pltpu.VMEM_SHARED((NPAD, DF), jnp.float32),
        ],
    )(dst3, zeros128, ones128)


def _spmm_body(y_hbm, e_hbm, zeros_hbm, out_hbm, ering, rows2, acc_sh, isem, gsem):
    c = lax.axis_index("c")
    s = lax.axis_index("s")
    ncg = jnp.where(c == 0, G0, G1)
    base = jnp.where(c == 0, s * G0, NSUB * G0 + s * G1)
    pltpu.sync_copy(zeros_hbm.at[pl.ds(s * SPT, SPT)], acc_sh.at[pl.ds(s * SPT, SPT)])
    for k in range(3):
        @pl.when(k < ncg)
        def _():
            pltpu.async_copy(e_hbm.at[base + k], ering.at[k], isem)
    for b in range(2):
        @pl.when(b < ncg)
        def _():
            pltpu.make_async_copy(e_hbm.at[base], ering.at[b], isem).wait()
            pltpu.async_copy(y_hbm.at[ering.at[b, 0]], rows2.at[b], gsem)
    plsc.subcore_barrier()

    def body(i, carry):
        b = lax.rem(i, 2)
        k = lax.rem(i, 4)
        pltpu.make_async_copy(y_hbm.at[pl.ds(0, CH)], rows2.at[b], gsem).wait()
        pltpu.sync_copy(rows2.at[b], acc_sh.at[ering.at[k, 1]], add=True)

        @pl.when(i + 3 < ncg)
        def _():
            pltpu.async_copy(e_hbm.at[base + i + 3], ering.at[lax.rem(i + 3, 4)], isem)

        @pl.when(i + 2 < ncg)
        def _():
            k2 = lax.rem(i + 2, 4)
            pltpu.make_async_copy(e_hbm.at[base], ering.at[k2], isem).wait()
            pltpu.async_copy(y_hbm.at[ering.at[k2, 0]], rows2.at[b], gsem)

        return carry

    lax.fori_loop(0, ncg, body, 0)
    plsc.subcore_barrier()
    pltpu.sync_copy(acc_sh.at[pl.ds(s * SPT, SPT)], out_hbm.at[c, pl.ds(s * SPT, SPT)])


def _sc_spmm(y, edges4, zeros128):
    return pl.kernel(
        _spmm_body,
        out_type=jax.ShapeDtypeStruct((2, NPAD, DF), jnp.float32),
        mesh=_sc_mesh(),
        scratch_types=[
            pltpu.VMEM((4, 2, CH), jnp.int32),
            pltpu.VMEM((2, CH, DF), jnp.float32),
            pltpu.VMEM_SHARED((NPAD, DF), jnp.float32),
            pltpu.SemaphoreType.DMA,
            pltpu.SemaphoreType.DMA,
        ],
    )(y, edges4, zeros128)


def _xw1_body(tac_ref, x_ref, w1_ref, emb_ref, o_ref):
    t = tac_ref[0]
    onehot = (lax.broadcasted_iota(jnp.int32, (1, 16), 1) == t).astype(jnp.float32)
    temb = jnp.dot(onehot, emb_ref[...], preferred_element_type=jnp.float32)
    c1 = jnp.dot(temb, w1_ref[pl.ds(DF, 16), :], preferred_element_type=jnp.float32)
    xw = jnp.dot(x_ref[...], w1_ref[pl.ds(0, DF), :], preferred_element_type=jnp.float32)
    o_ref[...] = xw + c1


def _tc_xw1(tac, x, W1, emb_table):
    return pl.pallas_call(
        _xw1_body,
        grid=(GRID,),
        in_specs=[
            pl.BlockSpec(memory_space=pltpu.MemorySpace.SMEM),
            pl.BlockSpec((BR, DF), lambda i: (i, 0)),
            pl.BlockSpec((DF + 16, DF), lambda i: (0, 0)),
            pl.BlockSpec((16, 16), lambda i: (0, 0)),
        ],
        out_specs=pl.BlockSpec((BR, DF), lambda i: (i, 0)),
        out_shape=jax.ShapeDtypeStruct((N, DF), jnp.float32),
    )(tac, x, W1, emb_table)


def _scale_body(deg_ref, xw_ref, y_ref, dinv_ref):
    d = deg_ref[0, :, 0:1] + deg_ref[1, :, 0:1] + 1.0
    dinv = lax.rsqrt(d)
    y_ref[...] = xw_ref[...] * dinv
    dinv_ref[...] = dinv


def _tc_scale(deg_p, xw1):
    return pl.pallas_call(
        _scale_body,
        grid=(GRID,),
        in_specs=[
            pl.BlockSpec((2, BR, DF), lambda i: (0, i, 0)),
            pl.BlockSpec((BR, DF), lambda i: (i, 0)),
        ],
        out_specs=[
            pl.BlockSpec((BR, DF), lambda i: (i, 0)),
            pl.BlockSpec((BR, 1), lambda i: (i, 0)),
        ],
        out_shape=[
            jax.ShapeDtypeStruct((N, DF), jnp.float32),
            jax.ShapeDtypeStruct((N, 1), jnp.float32),
        ],
    )(deg_p, xw1)


def _mid_body(p_ref, y_ref, dinv_ref, b1_ref, w2_ref, o_ref):
    sres = p_ref[0] + p_ref[1] + y_ref[...]
    h1 = jnp.maximum(sres * dinv_ref[...] + b1_ref[...], 0.0)
    xw2 = jnp.dot(h1, w2_ref[...], preferred_element_type=jnp.float32)
    o_ref[...] = xw2 * dinv_ref[...]


def _tc_mid(p1, y1, dinv, b1r, W2):
    return pl.pallas_call(
        _mid_body,
        grid=(GRID,),
        in_specs=[
            pl.BlockSpec((2, BR, DF), lambda i: (0, i, 0)),
            pl.BlockSpec((BR, DF), lambda i: (i, 0)),
            pl.BlockSpec((BR, 1), lambda i: (i, 0)),
            pl.BlockSpec((1, DF), lambda i: (0, 0)),
            pl.BlockSpec((DF, DF), lambda i: (0, 0)),
        ],
        out_specs=pl.BlockSpec((BR, DF), lambda i: (i, 0)),
        out_shape=jax.ShapeDtypeStruct((N, DF), jnp.float32),
    )(p1, y1, dinv, b1r, W2)


def _head_body(p_ref, y_ref, dinv_ref, b2_ref, wa_ref, ba_ref, wc_ref, bc_ref,
               ls_ref, mean_ref, std_ref, val_ref):
    sres = p_ref[0] + p_ref[1] + y_ref[...]
    h2 = jnp.maximum(sres * dinv_ref[...] + b2_ref[...], 0.0)
    mean_ref[...] = jnp.tanh(
        jnp.dot(h2, wa_ref[...], preferred_element_type=jnp.float32) + ba_ref[...])
    std_ref[...] = jnp.broadcast_to(jnp.exp(ls_ref[...]), (BR, 8))
    val_ref[...] = jnp.dot(h2, wc_ref[...], preferred_element_type=jnp.float32) + bc_ref[...]


def _tc_head(p2, y2, dinv, b2r, Wa, bar, Wc, bcr, log_std):
    return pl.pallas_call(
        _head_body,
        grid=(GRID,),
        in_specs=[
            pl.BlockSpec((2, BR, DF), lambda i: (0, i, 0)),
            pl.BlockSpec((BR, DF), lambda i: (i, 0)),
            pl.BlockSpec((BR, 1), lambda i: (i, 0)),
            pl.BlockSpec((1, DF), lambda i: (0, 0)),
            pl.BlockSpec((DF, 8), lambda i: (0, 0)),
            pl.BlockSpec((1, 8), lambda i: (0, 0)),
            pl.BlockSpec((DF, 1), lambda i: (0, 0)),
            pl.BlockSpec((1, 1), lambda i: (0, 0)),
            pl.BlockSpec((1, 8), lambda i: (0, 0)),
        ],
        out_specs=[
            pl.BlockSpec((BR, 8), lambda i: (i, 0)),
            pl.BlockSpec((BR, 8), lambda i: (i, 0)),
            pl.BlockSpec((BR, 1), lambda i: (i, 0)),
        ],
        out_shape=[
            jax.ShapeDtypeStruct((N, 8), jnp.float32),
            jax.ShapeDtypeStruct((N, 8), jnp.float32),
            jax.ShapeDtypeStruct((N, 1), jnp.float32),
        ],
    )(p2, y2, dinv, b2r, Wa, bar, Wc, bcr, log_std)


def kernel(x, edge_index, batch, tactic, emb_table, W1, b1, W2, b2, Wa, ba,
           log_std, Wc, bc):
    del batch
    src = edge_index[0].astype(jnp.int32)
    dst = edge_index[1].astype(jnp.int32)
    padn = EPAD - E
    src_p = jnp.concatenate([src, jnp.zeros((padn,), jnp.int32)])
    dst_p = jnp.concatenate([dst, jnp.full((padn,), DUMMY, jnp.int32)])
    dst3 = dst_p.reshape(NW, NCH, CH)
    tac = tactic.astype(jnp.int32).reshape(1)
    ones128 = jnp.ones((CH, DF), jnp.float32)
    zeros128 = jnp.zeros((NPAD, DF), jnp.float32)

    edges4 = jnp.stack([src_p.reshape(TOTCH, CH), dst_p.reshape(TOTCH, CH)],
                       axis=1)

    deg_p = _sc_degree(dst3, zeros128, ones128)
    xw1 = _tc_xw1(tac, x, W1, emb_table)
    y1, dinv = _tc_scale(deg_p, xw1)
    p1 = _sc_spmm(y1, edges4, zeros128)
    y2 = _tc_mid(p1, y1, dinv, b1.reshape(1, DF), W2)
    p2 = _sc_spmm(y2, edges4, zeros128)
    return _tc_head(p2, y2, dinv, b2.reshape(1, DF), Wa, ba.reshape(1, 8),
                    Wc, bc.reshape(1, 1), log_std)


# speedup vs baseline: 1.0033x; 1.0033x over previous
import functools

import jax
import jax.numpy as jnp
from jax import lax
from jax.experimental import pallas as pl
from jax.experimental.pallas import tpu as pltpu
from jax.experimental.pallas import tpu_sc as plsc

N = 10000
E = 320000
DF = 128
NPAD = 10240
EPAD = 327680
NW = 32
NSUB = 16
CH = 128
EPW = EPAD // NW
NCH = EPW // CH
TOTCH = EPAD // CH
G0 = 146
G1 = 14
SPT = NPAD // NSUB
DUMMY = 10200
BR = 1000
GRID = N // BR


def _sc_mesh():
    return plsc.VectorSubcoreMesh(core_axis_name="c", subcore_axis_name="s")


def _deg_body(dst_hbm, zeros_hbm, ones_hbm, out_hbm, didx2, ones_v, deg_sh):
    c = lax.axis_index("c")
    s = lax.axis_index("s")
    wid = c * NSUB + s
    pltpu.sync_copy(zeros_hbm.at[pl.ds(s * SPT, SPT)], deg_sh.at[pl.ds(s * SPT, SPT)])
    pltpu.sync_copy(dst_hbm.at[wid], didx2)
    pltpu.sync_copy(ones_hbm.at[pl.ds(0, CH)], ones_v)
    plsc.subcore_barrier()

    def body(i, carry):
        pltpu.sync_copy(ones_v, deg_sh.at[didx2.at[i]], add=True)
        return carry

    lax.fori_loop(0, NCH, body, 0)
    plsc.subcore_barrier()
    pltpu.sync_copy(deg_sh.at[pl.ds(s * SPT, SPT)], out_hbm.at[c, pl.ds(s * SPT, SPT)])


def _sc_degree(dst3, zeros128, ones128):
    return pl.kernel(
        _deg_body,
        out_type=jax.ShapeDtypeStruct((2, NPAD, DF), jnp.float32),
        mesh=_sc_mesh(),
        scratch_types=[
            pltpu.VMEM((NCH, CH), jnp.int32),
            pltpu.VMEM((CH, DF), jnp.float32),
            pltpu.VMEM_SHARED((NPAD, DF), jnp.float32),
        ],
    )(dst3, zeros128, ones128)


def _spmm_body(y_hbm, e_hbm, zeros_hbm, out_hbm, ering, rows2, acc_sh, isem, gsem):
    c = lax.axis_index("c")
    s = lax.axis_index("s")
    ncg = jnp.where(c == 0, G0, G1)
    base = jnp.where(c == 0, s * G0, NSUB * G0 + s * G1)
    pltpu.sync_copy(zeros_hbm.at[pl.ds(s * SPT, SPT)], acc_sh.at[pl.ds(s * SPT, SPT)])
    for k in range(3):
        @pl.when(k < ncg)
        def _():
            pltpu.async_copy(e_hbm.at[base + k], ering.at[k], isem)
    for b in range(2):
        @pl.when(b < ncg)
        def _():
            pltpu.make_async_copy(e_hbm.at[base], ering.at[b], isem).wait()
            pltpu.async_copy(y_hbm.at[ering.at[b, 0]], rows2.at[b], gsem)
    plsc.subcore_barrier()

    def body(i, carry):
        b = lax.rem(i, 2)
        k = lax.rem(i, 4)
        pltpu.make_async_copy(y_hbm.at[pl.ds(0, CH)], rows2.at[b], gsem).wait()
        pltpu.sync_copy(rows2.at[b], acc_sh.at[ering.at[k, 1]], add=True)

        @pl.when(i + 3 < ncg)
        def _():
            pltpu.async_copy(e_hbm.at[base + i + 3], ering.at[lax.rem(i + 3, 4)], isem)

        @pl.when(i + 2 < ncg)
        def _():
            k2 = lax.rem(i + 2, 4)
            pltpu.make_async_copy(e_hbm.at[base], ering.at[k2], isem).wait()
            pltpu.async_copy(y_hbm.at[ering.at[k2, 0]], rows2.at[b], gsem)

        return carry

    lax.fori_loop(0, ncg, body, 0)
    plsc.subcore_barrier()
    pltpu.sync_copy(acc_sh.at[pl.ds(s * SPT, SPT)], out_hbm.at[c, pl.ds(s * SPT, SPT)])


def _sc_spmm(y, edges4, zeros128):
    return pl.kernel(
        _spmm_body,
        out_type=jax.ShapeDtypeStruct((2, NPAD, DF), jnp.float32),
        mesh=_sc_mesh(),
        scratch_types=[
            pltpu.VMEM((4, 2, CH), jnp.int32),
            pltpu.VMEM((2, CH, DF), jnp.float32),
            pltpu.VMEM_SHARED((NPAD, DF), jnp.float32),
            pltpu.SemaphoreType.DMA,
            pltpu.SemaphoreType.DMA,
        ],
    )(y, edges4, zeros128)


def _xw1_body(tac_ref, x_ref, w1_ref, emb_ref, o_ref):
    t = tac_ref[0]
    onehot = (lax.broadcasted_iota(jnp.int32, (1, 16), 1) == t).astype(jnp.float32)
    temb = jnp.dot(onehot, emb_ref[...], preferred_element_type=jnp.float32)
    c1 = jnp.dot(temb, w1_ref[pl.ds(DF, 16), :], preferred_element_type=jnp.float32)
    xw = jnp.dot(x_ref[...], w1_ref[pl.ds(0, DF), :], preferred_element_type=jnp.float32)
    o_ref[...] = xw + c1


def _tc_xw1(tac, x, W1, emb_table):
    return pl.pallas_call(
        _xw1_body,
        grid=(GRID,),
        in_specs=[
            pl.BlockSpec(memory_space=pltpu.MemorySpace.SMEM),
            pl.BlockSpec((BR, DF), lambda i: (i, 0)),
            pl.BlockSpec((DF + 16, DF), lambda i: (0, 0)),
            pl.BlockSpec((16, 16), lambda i: (0, 0)),
        ],
        out_specs=pl.BlockSpec((BR, DF), lambda i: (i, 0)),
        out_shape=jax.ShapeDtypeStruct((N, DF), jnp.float32),
    )(tac, x, W1, emb_table)


def _scale_body(deg_ref, xw_ref, y_ref, dinv_ref):
    d = deg_ref[0, :, 0:1] + deg_ref[1, :, 0:1] + 1.0
    dinv = lax.rsqrt(d)
    y_ref[...] = xw_ref[...] * dinv
    dinv_ref[...] = dinv


def _tc_scale(deg_p, xw1):
    return pl.pallas_call(
        _scale_body,
        grid=(GRID,),
        in_specs=[
            pl.BlockSpec((2, BR, DF), lambda i: (0, i, 0)),
            pl.BlockSpec((BR, DF), lambda i: (i, 0)),
        ],
        out_specs=[
            pl.BlockSpec((BR, DF), lambda i: (i, 0)),
            pl.BlockSpec((BR, 1), lambda i: (i, 0)),
        ],
        out_shape=[
            jax.ShapeDtypeStruct((N, DF), jnp.float32),
            jax.ShapeDtypeStruct((N, 1), jnp.float32),
        ],
    )(deg_p, xw1)


def _mid_body(p_ref, y_ref, dinv_ref, b1_ref, w2_ref, o_ref):
    sres = p_ref[0] + p_ref[1] + y_ref[...]
    h1 = jnp.maximum(sres * dinv_ref[...] + b1_ref[...], 0.0)
    xw2 = jnp.dot(h1, w2_ref[...], preferred_element_type=jnp.float32)
    o_ref[...] = xw2 * dinv_ref[...]


def _tc_mid(p1, y1, dinv, b1r, W2):
    return pl.pallas_call(
        _mid_body,
        grid=(GRID,),
        in_specs=[
            pl.BlockSpec((2, BR, DF), lambda i: (0, i, 0)),
            pl.BlockSpec((BR, DF), lambda i: (i, 0)),
            pl.BlockSpec((BR, 1), lambda i: (i, 0)),
            pl.BlockSpec((1, DF), lambda i: (0, 0)),
            pl.BlockSpec((DF, DF), lambda i: (0, 0)),
        ],
        out_specs=pl.BlockSpec((BR, DF), lambda i: (i, 0)),
        out_shape=jax.ShapeDtypeStruct((N, DF), jnp.float32),
    )(p1, y1, dinv, b1r, W2)


def _head_body(p_ref, y_ref, dinv_ref, b2_ref, wa_ref, ba_ref, wc_ref, bc_ref,
               ls_ref, mean_ref, std_ref, val_ref):
    sres = p_ref[0] + p_ref[1] + y_ref[...]
    h2 = jnp.maximum(sres * dinv_ref[...] + b2_ref[...], 0.0)
    mean_ref[...] = jnp.tanh(
        jnp.dot(h2, wa_ref[...], preferred_element_type=jnp.float32) + ba_ref[...])
    std_ref[...] = jnp.broadcast_to(jnp.exp(ls_ref[...]), (BR, 8))
    val_ref[...] = jnp.dot(h2, wc_ref[...], preferred_element_type=jnp.float32) + bc_ref[...]


def _tc_head(p2, y2, dinv, b2r, Wa, bar, Wc, bcr, log_std):
    return pl.pallas_call(
        _head_body,
        grid=(GRID,),
        in_specs=[
            pl.BlockSpec((2, BR, DF), lambda i: (0, i, 0)),
            pl.BlockSpec((BR, DF), lambda i: (i, 0)),
            pl.BlockSpec((BR, 1), lambda i: (i, 0)),
            pl.BlockSpec((1, DF), lambda i: (0, 0)),
            pl.BlockSpec((DF, 8), lambda i: (0, 0)),
            pl.BlockSpec((1, 8), lambda i: (0, 0)),
            pl.BlockSpec((DF, 1), lambda i: (0, 0)),
            pl.BlockSpec((1, 1), lambda i: (0, 0)),
            pl.BlockSpec((1, 8), lambda i: (0, 0)),
        ],
        out_specs=[
            pl.BlockSpec((BR, 8), lambda i: (i, 0)),
            pl.BlockSpec((BR, 8), lambda i: (i, 0)),
            pl.BlockSpec((BR, 1), lambda i: (i, 0)),
        ],
        out_shape=[
            jax.ShapeDtypeStruct((N, 8), jnp.float32),
            jax.ShapeDtypeStruct((N, 8), jnp.float32),
            jax.ShapeDtypeStruct((N, 1), jnp.float32),
        ],
    )(p2, y2, dinv, b2r, Wa, bar, Wc, bcr, log_std)


def kernel(x, edge_index, batch, tactic, emb_table, W1, b1, W2, b2, Wa, ba,
           log_std, Wc, bc):
    del batch
    src = edge_index[0].astype(jnp.int32)
    dst = edge_index[1].astype(jnp.int32)
    padn = EPAD - E
    src_p = jnp.concatenate([src, jnp.zeros((padn,), jnp.int32)])
    dst_p = jnp.concatenate([dst, jnp.full((padn,), DUMMY, jnp.int32)])
    dst3 = dst_p.reshape(NW, NCH, CH)
    tac = tactic.astype(jnp.int32).reshape(1)
    ones128 = jnp.ones((CH, DF), jnp.float32)
    zeros128 = jnp.zeros((NPAD, DF), jnp.float32)

    edges4 = jnp.stack([src_p.reshape(TOTCH, CH), dst_p.reshape(TOTCH, CH)],
                       axis=1)

    deg_p = _sc_degree(dst3, zeros128, ones128)
    xw1 = _tc_xw1(tac, x, W1, emb_table)
    y1, dinv = _tc_scale(deg_p, xw1)
    p1 = _sc_spmm(y1, edges4, zeros128)
    y2 = _tc_mid(p1, y1, dinv, b1.reshape(1, DF), W2)
    p2 = _sc_spmm(y2, edges4, zeros128)
    return _tc_head(p2, y2, dinv, b2.reshape(1, DF), Wa, ba.reshape(1, 8),
                    Wc, bc.reshape(1, 1), log_std)
